# baseline (speedup 1.0000x reference)
import functools

import jax
import jax.numpy as jnp
from jax.experimental import pallas as pl
from jax.experimental.pallas import tpu as pltpu

_CH = 256
_NBUF = 8


def _make_body(m, k, d, ch, nbuf):
    nchunks = m // ch

    def _body(idx_ref, w_ref, a_hbm, x_hbm, o_hbm, xv, bufs, obufs,
              sem_x, sem_in, sem_out):
        idx = idx_ref[0]
        w = w_ref[idx]

        def a_copy(b, i):
            return pltpu.make_async_copy(
                a_hbm.at[idx, pl.ds(i * ch, ch), :], bufs.at[b], sem_in.at[b])

        def o_copy(b, i):
            return pltpu.make_async_copy(
                obufs.at[b], o_hbm.at[pl.ds(i * ch, ch), :], sem_out.at[b])

        pltpu.make_async_copy(x_hbm, xv, sem_x).start()
        for b in range(min(nbuf, nchunks)):
            a_copy(b, b).start()
        pltpu.make_async_copy(x_hbm, xv, sem_x).wait()

        for i in range(nchunks):
            b = i % nbuf
            a_copy(b, i).wait()
            if i >= nbuf:
                o_copy(b, i - nbuf).wait()
            obufs[b, :, :] = w * jnp.dot(
                bufs[b], xv[...], preferred_element_type=jnp.float32)
            o_copy(b, i).start()
            if i + nbuf < nchunks:
                a_copy(b, i + nbuf).start()

        for i in range(max(0, nchunks - nbuf), nchunks):
            o_copy(i % nbuf, i).wait()

    return _body


@functools.partial(jax.jit, static_argnames=("ch", "nbuf"))
def _conv(x, weight, adjs, idx, ch=_CH, nbuf=_NBUF):
    n, m, k = adjs.shape
    _, d = x.shape
    idx_arr = jnp.asarray(idx, jnp.int32).reshape((1,))
    grid_spec = pltpu.PrefetchScalarGridSpec(
        num_scalar_prefetch=2,
        grid=(1,),
        in_specs=[
            pl.BlockSpec(memory_space=pltpu.HBM),
            pl.BlockSpec(memory_space=pltpu.HBM),
        ],
        out_specs=pl.BlockSpec(memory_space=pltpu.HBM),
        scratch_shapes=[
            pltpu.VMEM((k, d), jnp.float32),
            pltpu.VMEM((nbuf, ch, k), jnp.float32),
            pltpu.VMEM((nbuf, ch, d), jnp.float32),
            pltpu.SemaphoreType.DMA,
            pltpu.SemaphoreType.DMA((nbuf,)),
            pltpu.SemaphoreType.DMA((nbuf,)),
        ],
    )
    return pl.pallas_call(
        _make_body(m, k, d, ch, nbuf),
        grid_spec=grid_spec,
        out_shape=jax.ShapeDtypeStruct((m, d), jnp.float32),
    )(idx_arr, weight, adjs, x)


def kernel(x, weight, adjs, idx):
    return _conv(x, weight, adjs, idx)


# speedup vs baseline: 1.2269x; 1.2269x over previous
import functools

import jax
import jax.numpy as jnp
from jax.experimental import pallas as pl
from jax.experimental.pallas import tpu as pltpu

_BM = 512


def _body(idx_ref, w_ref, a_ref, x_ref, o_ref):
    w = w_ref[idx_ref[0]]
    i = pl.program_id(0)
    acc = jnp.dot(a_ref[0], x_ref[...], preferred_element_type=jnp.float32)
    o_ref[pl.ds(i * _BM, _BM), :] = w * acc


@functools.partial(jax.jit, static_argnames=("bm",))
def _conv(x, weight, adjs, idx, bm=_BM):
    n, m, k = adjs.shape
    _, d = x.shape
    idx_arr = jnp.asarray(idx, jnp.int32).reshape((1,))
    grid_spec = pltpu.PrefetchScalarGridSpec(
        num_scalar_prefetch=2,
        grid=(m // bm,),
        in_specs=[
            pl.BlockSpec((1, bm, k), lambda i, idx_ref, w_ref: (idx_ref[0], i, 0)),
            pl.BlockSpec((k, d), lambda i, idx_ref, w_ref: (0, 0)),
        ],
        out_specs=pl.BlockSpec((m, d), lambda i, idx_ref, w_ref: (0, 0)),
    )
    return pl.pallas_call(
        _body,
        grid_spec=grid_spec,
        out_shape=jax.ShapeDtypeStruct((m, d), jnp.float32),
    )(idx_arr, weight, adjs, x)


def kernel(x, weight, adjs, idx):
    return _conv(x, weight, adjs, idx)
